# baseline (speedup 1.0000x reference)
import jax
import jax.numpy as jnp
from jax.experimental import pallas as pl
from jax.experimental.pallas import tpu as pltpu

_VMEM_LIMIT_BYTES = 64 * 1024 * 1024
_RANK_PAD = 8


def _lowrank_kernel(aw_ref, bw_ref, x_ref, o_ref):
    z = jnp.dot(bw_ref[...], x_ref[...], preferred_element_type=jnp.float32)
    o_ref[...] = jnp.dot(
        aw_ref[...], z, preferred_element_type=jnp.float32
    ).astype(o_ref.dtype)


def kernel(x, aw, bw):
    B, L, C = x.shape
    P, R = aw.shape
    itemsize = jnp.dtype(x.dtype).itemsize

    rp = max(_RANK_PAD, R)
    awp = jnp.zeros((P, rp), x.dtype).at[:, :R].set(aw.astype(x.dtype))
    bwp = jnp.zeros((rp, L), x.dtype).at[:R, :].set(bw.astype(x.dtype))

    tc = min(C, 2048)
    tc = max(128, (tc // 128) * 128) if C >= 128 else C
    grid = (B, pl.cdiv(C, tc))

    cost = pl.CostEstimate(
        flops=2 * (R * L + P * R) * B * C,
        transcendentals=0,
        bytes_accessed=(L + P) * B * C * itemsize,
    )

    return pl.pallas_call(
        _lowrank_kernel,
        out_shape=jax.ShapeDtypeStruct((B, P, C), x.dtype),
        grid_spec=pltpu.PrefetchScalarGridSpec(
            num_scalar_prefetch=0,
            grid=grid,
            in_specs=[
                pl.BlockSpec((P, rp), lambda b, j: (0, 0)),
                pl.BlockSpec((rp, L), lambda b, j: (0, 0)),
                pl.BlockSpec((None, L, tc), lambda b, j: (b, 0, j)),
            ],
            out_specs=pl.BlockSpec((None, P, tc), lambda b, j: (b, 0, j)),
        ),
        compiler_params=pltpu.CompilerParams(
            dimension_semantics=("parallel", "parallel"),
            vmem_limit_bytes=_VMEM_LIMIT_BYTES,
        ),
        cost_estimate=cost,
    )(awp, bwp, x)


# speedup vs baseline: 2.0186x; 2.0186x over previous
import jax
import jax.numpy as jnp
from jax.experimental import pallas as pl
from jax.experimental.pallas import tpu as pltpu

_VMEM_LIMIT_BYTES = 64 * 1024 * 1024
_RANK_PAD = 8


def _lowrank_kernel(aw_ref, bw_ref, x_ref, o_ref):
    nb = x_ref.shape[0]
    for i in range(nb):
        z = jnp.dot(bw_ref[...], x_ref[i], preferred_element_type=jnp.float32)
        o_ref[i] = jnp.dot(
            aw_ref[...], z, preferred_element_type=jnp.float32
        ).astype(o_ref.dtype)


def kernel(x, aw, bw):
    B, L, C = x.shape
    P, R = aw.shape
    itemsize = jnp.dtype(x.dtype).itemsize

    rp = max(_RANK_PAD, R)
    awp = jnp.zeros((P, rp), x.dtype).at[:, :R].set(aw.astype(x.dtype))
    bwp = jnp.zeros((rp, L), x.dtype).at[:R, :].set(bw.astype(x.dtype))

    tc = min(C, 2048)
    tc = max(128, (tc // 128) * 128) if C >= 128 else C
    nb = 1
    for cand in (8, 4, 2):
        if B % cand == 0:
            nb = cand
            break
    grid = (B // nb, pl.cdiv(C, tc))

    cost = pl.CostEstimate(
        flops=2 * (R * L + P * R) * B * C,
        transcendentals=0,
        bytes_accessed=(L + P) * B * C * itemsize,
    )

    return pl.pallas_call(
        _lowrank_kernel,
        out_shape=jax.ShapeDtypeStruct((B, P, C), x.dtype),
        grid_spec=pltpu.PrefetchScalarGridSpec(
            num_scalar_prefetch=0,
            grid=grid,
            in_specs=[
                pl.BlockSpec((P, rp), lambda b, j: (0, 0)),
                pl.BlockSpec((rp, L), lambda b, j: (0, 0)),
                pl.BlockSpec((nb, L, tc), lambda b, j: (b, 0, j)),
            ],
            out_specs=pl.BlockSpec((nb, P, tc), lambda b, j: (b, 0, j)),
        ),
        compiler_params=pltpu.CompilerParams(
            dimension_semantics=("parallel", "parallel"),
            vmem_limit_bytes=_VMEM_LIMIT_BYTES,
        ),
        cost_estimate=cost,
    )(awp, bwp, x)


# speedup vs baseline: 2.0479x; 1.0145x over previous
import jax
import jax.numpy as jnp
from jax.experimental import pallas as pl
from jax.experimental.pallas import tpu as pltpu

_VMEM_LIMIT_BYTES = 64 * 1024 * 1024
_RANK_PAD = 8


def _lowrank_kernel(aw_ref, bw_ref, x_ref, o_ref):
    nb = x_ref.shape[0]
    for i in range(nb):
        z = jnp.dot(bw_ref[...], x_ref[i], preferred_element_type=jnp.float32)
        o_ref[i] = jnp.dot(
            aw_ref[...], z, preferred_element_type=jnp.float32
        ).astype(o_ref.dtype)


def kernel(x, aw, bw):
    B, L, C = x.shape
    P, R = aw.shape
    itemsize = jnp.dtype(x.dtype).itemsize

    rp = max(_RANK_PAD, R)
    awp = jnp.zeros((P, rp), x.dtype).at[:, :R].set(aw.astype(x.dtype))
    bwp = jnp.zeros((rp, L), x.dtype).at[:R, :].set(bw.astype(x.dtype))

    tc = min(C, 2048)
    tc = max(128, (tc // 128) * 128) if C >= 128 else C
    nb = 1
    for cand in (16, 8, 4, 2):
        if B % cand == 0:
            nb = cand
            break
    grid = (B // nb, pl.cdiv(C, tc))

    cost = pl.CostEstimate(
        flops=2 * (R * L + P * R) * B * C,
        transcendentals=0,
        bytes_accessed=(L + P) * B * C * itemsize,
    )

    return pl.pallas_call(
        _lowrank_kernel,
        out_shape=jax.ShapeDtypeStruct((B, P, C), x.dtype),
        grid_spec=pltpu.PrefetchScalarGridSpec(
            num_scalar_prefetch=0,
            grid=grid,
            in_specs=[
                pl.BlockSpec((P, rp), lambda b, j: (0, 0)),
                pl.BlockSpec((rp, L), lambda b, j: (0, 0)),
                pl.BlockSpec((nb, L, tc), lambda b, j: (b, 0, j)),
            ],
            out_specs=pl.BlockSpec((nb, P, tc), lambda b, j: (b, 0, j)),
        ),
        compiler_params=pltpu.CompilerParams(
            dimension_semantics=("parallel", "parallel"),
            vmem_limit_bytes=_VMEM_LIMIT_BYTES,
        ),
        cost_estimate=cost,
    )(awp, bwp, x)


# speedup vs baseline: 2.0582x; 1.0051x over previous
import jax
import jax.numpy as jnp
from jax.experimental import pallas as pl
from jax.experimental.pallas import tpu as pltpu

_VMEM_LIMIT_BYTES = 64 * 1024 * 1024
_RANK_PAD = 8


def _lowrank_kernel(aw_ref, bw_ref, x_ref, o_ref):
    nb = x_ref.shape[0]
    for i in range(nb):
        z = jnp.dot(bw_ref[...], x_ref[i], preferred_element_type=jnp.float32)
        o_ref[i] = jnp.dot(
            aw_ref[...], z, preferred_element_type=jnp.float32
        ).astype(o_ref.dtype)


def kernel(x, aw, bw):
    B, L, C = x.shape
    P, R = aw.shape
    itemsize = jnp.dtype(x.dtype).itemsize

    rp = max(_RANK_PAD, R)
    awp = jnp.zeros((P, rp), x.dtype).at[:, :R].set(aw.astype(x.dtype))
    bwp = jnp.zeros((rp, L), x.dtype).at[:R, :].set(bw.astype(x.dtype))

    tc = min(C, 2048)
    tc = max(128, (tc // 128) * 128) if C >= 128 else C
    nb = 1
    for cand in (16, 8, 4, 2):
        if B % cand == 0:
            nb = cand
            break
    cost = pl.CostEstimate(
        flops=2 * (R * L + P * R) * B * C,
        transcendentals=0,
        bytes_accessed=(L + P) * B * C * itemsize,
    )

    if tc == C:
        grid = (B // nb,)
        in_specs = [
            pl.BlockSpec((P, rp), lambda b: (0, 0)),
            pl.BlockSpec((rp, L), lambda b: (0, 0)),
            pl.BlockSpec((nb, L, tc), lambda b: (b, 0, 0)),
        ]
        out_specs = pl.BlockSpec((nb, P, tc), lambda b: (b, 0, 0))
        semantics = ("parallel",)
    else:
        grid = (B // nb, pl.cdiv(C, tc))
        in_specs = [
            pl.BlockSpec((P, rp), lambda b, j: (0, 0)),
            pl.BlockSpec((rp, L), lambda b, j: (0, 0)),
            pl.BlockSpec((nb, L, tc), lambda b, j: (b, 0, j)),
        ]
        out_specs = pl.BlockSpec((nb, P, tc), lambda b, j: (b, 0, j))
        semantics = ("parallel", "parallel")

    return pl.pallas_call(
        _lowrank_kernel,
        out_shape=jax.ShapeDtypeStruct((B, P, C), x.dtype),
        grid_spec=pltpu.PrefetchScalarGridSpec(
            num_scalar_prefetch=0,
            grid=grid,
            in_specs=in_specs,
            out_specs=out_specs,
        ),
        compiler_params=pltpu.CompilerParams(
            dimension_semantics=semantics,
            vmem_limit_bytes=_VMEM_LIMIT_BYTES,
        ),
        cost_estimate=cost,
    )(awp, bwp, x)
